# baseline (speedup 1.0000x reference)
import functools

import jax
import jax.numpy as jnp
from jax import lax
from jax.experimental import pallas as pl
from jax.experimental.pallas import tpu as pltpu
from jax.experimental.pallas import tpu_sc as plsc

N_NODES = 10000
N_EDGES = 320000
D_FEAT = 128
NC = 2
NS = 16
CH = 64
EP = -(-N_EDGES // (NC * NS * CH)) * (NC * NS * CH)
NCHUNK = EP // (NC * NS * CH)
ACC_R = 10240
ZR = ACC_R // NS
NBUF = 3
GAHEAD = 2
SLAG = 1


def _sc_body(x_h, ei_h, zr_h, out_h, dst_v, src_v, rows_v, acc, gsem, ssem):
    cid = lax.axis_index("c")
    sid = lax.axis_index("s")

    pltpu.async_copy(zr_h, acc.at[pl.ds(sid * ZR, ZR)], gsem)
    pltpu.async_copy(ei_h.at[0, cid, sid], dst_v, gsem)
    pltpu.async_copy(ei_h.at[1, cid, sid], src_v, gsem)
    pltpu.make_async_copy(zr_h, acc.at[pl.ds(sid * ZR, ZR)], gsem).wait()
    pltpu.make_async_copy(ei_h.at[0, cid, sid], dst_v, gsem).wait()
    pltpu.make_async_copy(ei_h.at[1, cid, sid], src_v, gsem).wait()
    plsc.subcore_barrier()

    for k in range(GAHEAD):
        pltpu.async_copy(x_h.at[src_v.at[k]], rows_v.at[k], gsem)

    def chunk(j, carry):
        b = lax.rem(j, NBUF)
        pltpu.make_async_copy(x_h.at[src_v.at[j]], rows_v.at[b], gsem).wait()
        pltpu.async_copy(rows_v.at[b], acc.at[dst_v.at[j]], ssem, add=True)

        @pl.when(j >= SLAG)
        def _():
            pltpu.make_async_copy(rows_v.at[b], acc.at[dst_v.at[j]],
                                  ssem).wait()

        @pl.when(j < NCHUNK - GAHEAD)
        def _():
            pltpu.async_copy(x_h.at[src_v.at[j + GAHEAD]],
                             rows_v.at[lax.rem(j + GAHEAD, NBUF)], gsem)

        return carry

    lax.fori_loop(0, NCHUNK, chunk, 0)
    for _ in range(SLAG):
        pltpu.make_async_copy(rows_v.at[0], acc.at[dst_v.at[0]], ssem).wait()
    plsc.subcore_barrier()

    pltpu.sync_copy(acc.at[pl.ds(sid * ZR, ZR)],
                    out_h.at[cid, pl.ds(sid * ZR, ZR)])


@functools.partial(
    pl.kernel,
    out_type=jax.ShapeDtypeStruct((NC, ACC_R, D_FEAT), jnp.float32),
    mesh=plsc.VectorSubcoreMesh(core_axis_name="c", subcore_axis_name="s"),
    compiler_params=pltpu.CompilerParams(use_tc_tiling_on_sc=False),
    scratch_types=[
        pltpu.VMEM((NCHUNK, CH), jnp.int32),
        pltpu.VMEM((NCHUNK, CH), jnp.int32),
        pltpu.VMEM((NBUF, CH, D_FEAT), jnp.float32),
        pltpu.VMEM_SHARED((ACC_R, D_FEAT), jnp.float32),
        pltpu.SemaphoreType.DMA,
        pltpu.SemaphoreType.DMA,
    ],
)
def _mp_kernel(x_h, ei_h, zr_h, out_h, dst_v, src_v, rows_v, acc,
               gsem, ssem):
    _sc_body(x_h, ei_h, zr_h, out_h, dst_v, src_v, rows_v, acc, gsem, ssem)


_BLK = 1000


def _add_body(a_ref, b_ref, o_ref):
    o_ref[...] = a_ref[0] + b_ref[0]


_add_partials = functools.partial(
    pl.pallas_call,
    _add_body,
    out_shape=jax.ShapeDtypeStruct((N_NODES, D_FEAT), jnp.float32),
    grid=(N_NODES // _BLK,),
    in_specs=[
        pl.BlockSpec((1, _BLK, D_FEAT), lambda i: (0, i, 0)),
        pl.BlockSpec((1, _BLK, D_FEAT), lambda i: (1, i, 0)),
    ],
    out_specs=pl.BlockSpec((_BLK, D_FEAT), lambda i: (i, 0)),
)()


def kernel(X, edge_index):
    npad = EP - N_EDGES
    pad_dst = N_NODES + jnp.arange(npad, dtype=jnp.int32) % (ACC_R - N_NODES)
    pad = jnp.stack([pad_dst, jnp.zeros((npad,), jnp.int32)])
    eip = jnp.concatenate([edge_index, pad], axis=1)
    eip = eip.reshape(2, NC, NS, NCHUNK, CH)
    zrows = jnp.zeros((ZR, D_FEAT), jnp.float32)
    partials = _mp_kernel(X, eip, zrows)
    return _add_partials(partials, partials)


# speedup vs baseline: 1.2543x; 1.2543x over previous
import functools

import jax
import jax.numpy as jnp
from jax import lax
from jax.experimental import pallas as pl
from jax.experimental.pallas import tpu as pltpu
from jax.experimental.pallas import tpu_sc as plsc

N_NODES = 10000
N_EDGES = 320000
D_FEAT = 128
DH = D_FEAT // 2
NC = 2
NS = 16
CH = 128
EP = -(-N_EDGES // (NS * CH)) * (NS * CH)
NCHUNK = EP // (NS * CH)
ACC_R = 10240
ZR = ACC_R // NS
OR = N_NODES // NS
NBUF = 6
GAHEAD = 4
SLAG = 2


def _sc_body(x2_h, ei_h, zr_h, out_h, dst_v, src_v, rows_v, acc, gsem, ssem):
    cid = lax.axis_index("c")
    sid = lax.axis_index("s")

    pltpu.async_copy(zr_h, acc.at[pl.ds(sid * ZR, ZR)], gsem)
    pltpu.async_copy(ei_h.at[0, sid], dst_v, gsem)
    pltpu.async_copy(ei_h.at[1, sid], src_v, gsem)
    pltpu.make_async_copy(zr_h, acc.at[pl.ds(sid * ZR, ZR)], gsem).wait()
    pltpu.make_async_copy(ei_h.at[0, sid], dst_v, gsem).wait()
    pltpu.make_async_copy(ei_h.at[1, sid], src_v, gsem).wait()
    plsc.subcore_barrier()

    def remap(j):
        for k in range(CH // 16):
            s = src_v[j, pl.ds(k * 16, 16)]
            src_v[j, pl.ds(k * 16, 16)] = s + s + cid

    for k in range(GAHEAD):
        remap(k)
        pltpu.async_copy(x2_h.at[src_v.at[k]], rows_v.at[k], gsem)

    def chunk(j, carry):
        @pl.when(j < NCHUNK - GAHEAD)
        def _():
            remap(j + GAHEAD)

        b = lax.rem(j, NBUF)
        pltpu.make_async_copy(x2_h.at[src_v.at[j]], rows_v.at[b], gsem).wait()
        pltpu.async_copy(rows_v.at[b], acc.at[dst_v.at[j]], ssem, add=True)

        @pl.when(j >= SLAG)
        def _():
            pltpu.make_async_copy(rows_v.at[b], acc.at[dst_v.at[j]],
                                  ssem).wait()

        @pl.when(j < NCHUNK - GAHEAD)
        def _():
            pltpu.async_copy(x2_h.at[src_v.at[j + GAHEAD]],
                             rows_v.at[lax.rem(j + GAHEAD, NBUF)], gsem)

        return carry

    lax.fori_loop(0, NCHUNK, chunk, 0)
    for _ in range(SLAG):
        pltpu.make_async_copy(rows_v.at[0], acc.at[dst_v.at[0]], ssem).wait()
    plsc.subcore_barrier()

    pltpu.sync_copy(acc.at[pl.ds(sid * OR, OR)],
                    out_h.at[pl.ds(sid * OR, OR), pl.ds(cid * DH, DH)])


@functools.partial(
    pl.kernel,
    out_type=jax.ShapeDtypeStruct((N_NODES, D_FEAT), jnp.float32),
    mesh=plsc.VectorSubcoreMesh(core_axis_name="c", subcore_axis_name="s"),
    compiler_params=pltpu.CompilerParams(use_tc_tiling_on_sc=False),
    scratch_types=[
        pltpu.VMEM((NCHUNK, CH), jnp.int32),
        pltpu.VMEM((NCHUNK, CH), jnp.int32),
        pltpu.VMEM((NBUF, CH, DH), jnp.float32),
        pltpu.VMEM_SHARED((ACC_R, DH), jnp.float32),
        pltpu.SemaphoreType.DMA,
        pltpu.SemaphoreType.DMA,
    ],
)
def _mp_kernel(x2_h, ei_h, zr_h, out_h, dst_v, src_v, rows_v, acc,
               gsem, ssem):
    _sc_body(x2_h, ei_h, zr_h, out_h, dst_v, src_v, rows_v, acc, gsem, ssem)


def kernel(X, edge_index):
    npad = EP - N_EDGES
    pad_dst = N_NODES + jnp.arange(npad, dtype=jnp.int32) % (ACC_R - N_NODES)
    pad = jnp.stack([pad_dst, jnp.zeros((npad,), jnp.int32)])
    eip = jnp.concatenate([edge_index, pad], axis=1)
    eip = eip.reshape(2, NS, NCHUNK, CH)
    x2 = X.reshape(NC * N_NODES, DH)
    zrows = jnp.zeros((ZR, DH), jnp.float32)
    return _mp_kernel(x2, eip, zrows)


# speedup vs baseline: 1.2713x; 1.0135x over previous
import functools

import jax
import jax.numpy as jnp
import numpy as np
from jax import lax
from jax.experimental import pallas as pl
from jax.experimental.pallas import tpu as pltpu
from jax.experimental.pallas import tpu_sc as plsc

N_NODES = 10000
N_EDGES = 320000
D_FEAT = 128
DH = D_FEAT // 2
NC = 2
NS = 16
CH = 128
EP = -(-N_EDGES // (NS * CH)) * (NS * CH)
NCHUNK = EP // (NS * CH)
ACC_R = 10240
ZR = ACC_R // NS
OR = N_NODES // NS
NBUF = 6
GAHEAD = 4
SLAG = 2


def _sc_body(x2_h, ei_h, zr_h, out_h, dst_v, src_v, rows_v, acc, gsem, ssem):
    cid = lax.axis_index("c")
    sid = lax.axis_index("s")

    pltpu.async_copy(zr_h, acc.at[pl.ds(sid * ZR, ZR)], gsem)
    pltpu.async_copy(ei_h.at[0, sid], dst_v, gsem)
    pltpu.async_copy(ei_h.at[1, sid], src_v, gsem)
    pltpu.make_async_copy(zr_h, acc.at[pl.ds(sid * ZR, ZR)], gsem).wait()
    pltpu.make_async_copy(ei_h.at[0, sid], dst_v, gsem).wait()
    pltpu.make_async_copy(ei_h.at[1, sid], src_v, gsem).wait()
    plsc.subcore_barrier()

    def remap(j):
        for k in range(CH // 16):
            s = src_v[j, pl.ds(k * 16, 16)]
            src_v[j, pl.ds(k * 16, 16)] = s + s + cid

    for k in range(GAHEAD):
        remap(k)
        pltpu.async_copy(x2_h.at[src_v.at[k]], rows_v.at[k], gsem)

    def chunk(j, carry):
        @pl.when(j < NCHUNK - GAHEAD)
        def _():
            remap(j + GAHEAD)

        b = lax.rem(j, NBUF)
        pltpu.make_async_copy(x2_h.at[src_v.at[j]], rows_v.at[b], gsem).wait()
        pltpu.async_copy(rows_v.at[b], acc.at[dst_v.at[j]], ssem, add=True)

        @pl.when(j >= SLAG)
        def _():
            pltpu.make_async_copy(rows_v.at[b], acc.at[dst_v.at[j]],
                                  ssem).wait()

        @pl.when(j < NCHUNK - GAHEAD)
        def _():
            pltpu.async_copy(x2_h.at[src_v.at[j + GAHEAD]],
                             rows_v.at[lax.rem(j + GAHEAD, NBUF)], gsem)

        return carry

    lax.fori_loop(0, NCHUNK, chunk, 0)
    for _ in range(SLAG):
        pltpu.make_async_copy(rows_v.at[0], acc.at[dst_v.at[0]], ssem).wait()
    plsc.subcore_barrier()

    pltpu.sync_copy(acc.at[pl.ds(sid * OR, OR)],
                    out_h.at[pl.ds(sid * OR, OR), pl.ds(cid * DH, DH)])


@functools.partial(
    pl.kernel,
    out_type=jax.ShapeDtypeStruct((N_NODES, D_FEAT), jnp.float32),
    mesh=plsc.VectorSubcoreMesh(core_axis_name="c", subcore_axis_name="s"),
    compiler_params=pltpu.CompilerParams(use_tc_tiling_on_sc=False),
    scratch_types=[
        pltpu.VMEM((NCHUNK, CH), jnp.int32),
        pltpu.VMEM((NCHUNK, CH), jnp.int32),
        pltpu.VMEM((NBUF, CH, DH), jnp.float32),
        pltpu.VMEM_SHARED((ACC_R, DH), jnp.float32),
        pltpu.SemaphoreType.DMA,
        pltpu.SemaphoreType.DMA,
    ],
)
def _mp_kernel(x2_h, ei_h, zr_h, out_h, dst_v, src_v, rows_v, acc,
               gsem, ssem):
    _sc_body(x2_h, ei_h, zr_h, out_h, dst_v, src_v, rows_v, acc, gsem, ssem)


def kernel(X, edge_index):
    npad = EP - N_EDGES
    pad_dst = N_NODES + np.arange(npad, dtype=np.int32) % (ACC_R - N_NODES)
    pad = np.stack([pad_dst, np.zeros((npad,), np.int32)])
    eip = jnp.concatenate([edge_index, jnp.asarray(pad)], axis=1)
    eip = eip.reshape(2, NS, NCHUNK, CH)
    x2 = X.reshape(NC * N_NODES, DH)
    zrows = jnp.zeros((ZR, DH), jnp.float32)
    return _mp_kernel(x2, eip, zrows)
